# baseline (device time: 83352 ns/iter reference)
import jax
import jax.numpy as jnp
from jax import lax
from jax.experimental import pallas as pl
from jax.experimental.pallas import tpu as pltpu

N_DEV = 4
B = 2
SQ = 512
HQ_LOCAL = 8
DH = 64
D_MODEL = 768
ROWS = B * SQ
CHUNK = ROWS // N_DEV
WINDOW = 128
SCALE = 0.125


def kernel(x, Wq, K_ext, V_ext, Wo):
    my = lax.axis_index("i")
    h0 = my * HQ_LOCAL
    K = lax.dynamic_slice_in_dim(K_ext, h0, HQ_LOCAL, axis=2)
    V = lax.dynamic_slice_in_dim(V_ext, h0, HQ_LOCAL, axis=2)
    K = K.transpose(0, 2, 1, 3).reshape(B * HQ_LOCAL, SQ, DH)
    V = V.transpose(0, 2, 1, 3).reshape(B * HQ_LOCAL, SQ, DH)
    x2 = x.reshape(ROWS, D_MODEL)

    def body(x_ref, wq_ref, k_ref, v_ref, wo_ref, out_ref,
             part_ref, acc_ref, rbuf_ref,
             rs_send, rs_recv, ag_send, ag_recv):
        my_pos = lax.axis_index("i")
        left = (my_pos + N_DEV - 1) % N_DEV
        right = (my_pos + 1) % N_DEV

        barrier_sem = pltpu.get_barrier_semaphore()
        for nbr in (left, right):
            pl.semaphore_signal(
                barrier_sem, inc=1,
                device_id=(nbr,), device_id_type=pl.DeviceIdType.MESH,
            )
        pl.semaphore_wait(barrier_sem, 2)

        q_all = jnp.dot(x_ref[...], wq_ref[...],
                        preferred_element_type=jnp.float32)

        qi = lax.broadcasted_iota(jnp.int32, (SQ, SQ), 0)
        ki = lax.broadcasted_iota(jnp.int32, (SQ, SQ), 1)
        neg = jnp.where(jnp.abs(qi - ki) <= WINDOW, 0.0, -1e9).astype(jnp.float32)

        for b in range(B):
            for h in range(HQ_LOCAL):
                q = q_all[b * SQ:(b + 1) * SQ, h * DH:(h + 1) * DH]
                k = k_ref[b * HQ_LOCAL + h]
                v = v_ref[b * HQ_LOCAL + h]
                s = lax.dot_general(
                    q, k, (((1,), (1,)), ((), ())),
                    preferred_element_type=jnp.float32,
                ) * SCALE + neg
                m = jnp.max(s, axis=1, keepdims=True)
                w = jnp.exp(s - m)
                w = w / jnp.sum(w, axis=1, keepdims=True)
                ctx = jnp.dot(w, v, preferred_element_type=jnp.float32)
                acc_ref[b * SQ:(b + 1) * SQ, h * DH:(h + 1) * DH] = ctx

        part_ref[...] = jnp.dot(acc_ref[...], wo_ref[...],
                                preferred_element_type=jnp.float32)

        for s in range(N_DEV - 1):
            send_c = (my_pos + (N_DEV - s)) % N_DEV
            recv_c = (my_pos + (N_DEV - s - 1)) % N_DEV
            rdma = pltpu.make_async_remote_copy(
                src_ref=part_ref.at[pl.ds(send_c * CHUNK, CHUNK)],
                dst_ref=rbuf_ref.at[s],
                send_sem=rs_send.at[s],
                recv_sem=rs_recv.at[s],
                device_id=(right,),
                device_id_type=pl.DeviceIdType.MESH,
            )
            rdma.start()
            rdma.wait()
            part_ref[pl.ds(recv_c * CHUNK, CHUNK), :] += rbuf_ref[s]

        myc = (my_pos + 1) % N_DEV
        out_ref[pl.ds(myc * CHUNK, CHUNK), :] = part_ref[
            pl.ds(myc * CHUNK, CHUNK), :]

        for t in range(N_DEV - 1):
            send_c = (my_pos + (N_DEV + 1 - t)) % N_DEV
            rdma = pltpu.make_async_remote_copy(
                src_ref=out_ref.at[pl.ds(send_c * CHUNK, CHUNK)],
                dst_ref=out_ref.at[pl.ds(send_c * CHUNK, CHUNK)],
                send_sem=ag_send.at[t],
                recv_sem=ag_recv.at[t],
                device_id=(right,),
                device_id_type=pl.DeviceIdType.MESH,
            )
            rdma.start()
            rdma.wait()

    out = pl.pallas_call(
        body,
        out_shape=jax.ShapeDtypeStruct((ROWS, D_MODEL), jnp.float32),
        in_specs=[pl.BlockSpec(memory_space=pltpu.VMEM)] * 5,
        out_specs=pl.BlockSpec(memory_space=pltpu.VMEM),
        scratch_shapes=[
            pltpu.VMEM((ROWS, D_MODEL), jnp.float32),
            pltpu.VMEM((ROWS, HQ_LOCAL * DH), jnp.float32),
            pltpu.VMEM((N_DEV - 1, CHUNK, D_MODEL), jnp.float32),
            pltpu.SemaphoreType.DMA((N_DEV - 1,)),
            pltpu.SemaphoreType.DMA((N_DEV - 1,)),
            pltpu.SemaphoreType.DMA((N_DEV - 1,)),
            pltpu.SemaphoreType.DMA((N_DEV - 1,)),
        ],
        compiler_params=pltpu.CompilerParams(collective_id=0),
    )(x2, Wq, K, V, Wo)

    return out.reshape(B, SQ, D_MODEL)


# device time: 44844 ns/iter; 1.8587x vs baseline; 1.8587x over previous
import jax
import jax.numpy as jnp
from jax import lax
from jax.experimental import pallas as pl
from jax.experimental.pallas import tpu as pltpu

N_DEV = 4
B = 2
SQ = 512
HQ_LOCAL = 8
DH = 64
D_MODEL = 768
ROWS = B * SQ
CHUNK = ROWS // N_DEV
WINDOW = 128
SCALE = 0.125


def kernel(x, Wq, K_ext, V_ext, Wo):
    my = lax.axis_index("i")
    h0 = my * HQ_LOCAL
    K = lax.dynamic_slice_in_dim(K_ext, h0, HQ_LOCAL, axis=2)
    V = lax.dynamic_slice_in_dim(V_ext, h0, HQ_LOCAL, axis=2)
    K = K.transpose(0, 2, 1, 3).reshape(B * HQ_LOCAL, SQ, DH)
    V = V.transpose(0, 2, 1, 3).reshape(B * HQ_LOCAL, SQ, DH)
    x2 = x.reshape(ROWS, D_MODEL)

    def body(x_ref, wq_ref, k_ref, v_ref, wo_ref, out_ref,
             part_ref, part_bf_ref, acc_ref, rbuf_ref, agbuf_ref, out_bf_ref,
             rs_send, rs_recv, ag_send, ag_recv):
        my_pos = lax.axis_index("i")
        left = (my_pos + N_DEV - 1) % N_DEV
        right = (my_pos + 1) % N_DEV
        diag = (my_pos + 2) % N_DEV
        peers = (right, left, diag)

        barrier_sem = pltpu.get_barrier_semaphore()
        for nbr in peers:
            pl.semaphore_signal(
                barrier_sem, inc=1,
                device_id=(nbr,), device_id_type=pl.DeviceIdType.MESH,
            )
        pl.semaphore_wait(barrier_sem, 3)

        q_all = jnp.dot(x_ref[...], wq_ref[...],
                        preferred_element_type=jnp.float32)

        qi = lax.broadcasted_iota(jnp.int32, (SQ, SQ), 0)
        ki = lax.broadcasted_iota(jnp.int32, (SQ, SQ), 1)
        neg = jnp.where(jnp.abs(qi - ki) <= WINDOW, 0.0, -1e9).astype(jnp.float32)

        for b in range(B):
            for h in range(HQ_LOCAL):
                q = q_all[b * SQ:(b + 1) * SQ, h * DH:(h + 1) * DH]
                k = k_ref[b * HQ_LOCAL + h]
                v = v_ref[b * HQ_LOCAL + h]
                s = lax.dot_general(
                    q, k, (((1,), (1,)), ((), ())),
                    preferred_element_type=jnp.float32,
                ) * SCALE + neg
                m = jnp.max(s, axis=1, keepdims=True)
                w = jnp.exp(s - m)
                w = w / jnp.sum(w, axis=1, keepdims=True)
                ctx = jnp.dot(w, v, preferred_element_type=jnp.float32)
                acc_ref[b * SQ:(b + 1) * SQ, h * DH:(h + 1) * DH] = ctx

        part_ref[...] = jnp.dot(acc_ref[...], wo_ref[...],
                                preferred_element_type=jnp.float32)
        part_bf_ref[...] = part_ref[...].astype(jnp.bfloat16)

        rs = []
        for slot, p in enumerate(peers):
            rdma = pltpu.make_async_remote_copy(
                src_ref=part_bf_ref.at[pl.ds(p * CHUNK, CHUNK)],
                dst_ref=rbuf_ref.at[slot],
                send_sem=rs_send.at[slot],
                recv_sem=rs_recv.at[slot],
                device_id=(p,),
                device_id_type=pl.DeviceIdType.MESH,
            )
            rdma.start()
            rs.append(rdma)
        for rdma in rs:
            rdma.wait()

        red = part_ref[pl.ds(my_pos * CHUNK, CHUNK), :]
        for slot in range(3):
            red = red + rbuf_ref[slot].astype(jnp.float32)
        red_bf = red.astype(jnp.bfloat16)
        agbuf_ref[...] = red_bf
        out_bf_ref[pl.ds(my_pos * CHUNK, CHUNK), :] = red_bf

        ag = []
        for slot, p in enumerate(peers):
            rdma = pltpu.make_async_remote_copy(
                src_ref=agbuf_ref,
                dst_ref=out_bf_ref.at[pl.ds(my_pos * CHUNK, CHUNK)],
                send_sem=ag_send.at[slot],
                recv_sem=ag_recv.at[slot],
                device_id=(p,),
                device_id_type=pl.DeviceIdType.MESH,
            )
            rdma.start()
            ag.append(rdma)
        for rdma in ag:
            rdma.wait()

        out_ref[...] = out_bf_ref[...].astype(jnp.float32)

    out = pl.pallas_call(
        body,
        out_shape=jax.ShapeDtypeStruct((ROWS, D_MODEL), jnp.float32),
        in_specs=[pl.BlockSpec(memory_space=pltpu.VMEM)] * 5,
        out_specs=pl.BlockSpec(memory_space=pltpu.VMEM),
        scratch_shapes=[
            pltpu.VMEM((ROWS, D_MODEL), jnp.float32),
            pltpu.VMEM((ROWS, D_MODEL), jnp.bfloat16),
            pltpu.VMEM((ROWS, HQ_LOCAL * DH), jnp.float32),
            pltpu.VMEM((3, CHUNK, D_MODEL), jnp.bfloat16),
            pltpu.VMEM((CHUNK, D_MODEL), jnp.bfloat16),
            pltpu.VMEM((ROWS, D_MODEL), jnp.bfloat16),
            pltpu.SemaphoreType.DMA((3,)),
            pltpu.SemaphoreType.DMA((3,)),
            pltpu.SemaphoreType.DMA((3,)),
            pltpu.SemaphoreType.DMA((3,)),
        ],
        compiler_params=pltpu.CompilerParams(collective_id=0),
    )(x2, Wq, K, V, Wo)

    return out.reshape(B, SQ, D_MODEL)


# device time: 44316 ns/iter; 1.8809x vs baseline; 1.0119x over previous
import jax
import jax.numpy as jnp
from jax import lax
from jax.experimental import pallas as pl
from jax.experimental.pallas import tpu as pltpu

N_DEV = 4
B = 2
SQ = 512
HQ_LOCAL = 8
DH = 64
D_MODEL = 768
QDIM = HQ_LOCAL * DH
ROWS = B * SQ
CHUNK = ROWS // N_DEV
WINDOW = 128
SCALE = 0.125


def kernel(x, Wq, K_ext, V_ext, Wo):
    my = lax.axis_index("i")
    h0 = my * HQ_LOCAL
    K = lax.dynamic_slice_in_dim(K_ext, h0, HQ_LOCAL, axis=2)
    V = lax.dynamic_slice_in_dim(V_ext, h0, HQ_LOCAL, axis=2)
    K = K.transpose(0, 2, 1, 3).reshape(B * HQ_LOCAL, SQ, DH).astype(jnp.bfloat16)
    V = V.transpose(0, 2, 1, 3).reshape(B * HQ_LOCAL, SQ, DH).astype(jnp.bfloat16)
    x2 = x.reshape(ROWS, D_MODEL).astype(jnp.bfloat16)
    Wq_bf = Wq.astype(jnp.bfloat16)
    Wo_bf = Wo.astype(jnp.bfloat16)

    def body(x_ref, wq_ref, k_ref, v_ref, wo_ref, out_ref,
             q_ref, accc_ref, sbuf_ref, rbuf_ref, agbuf_ref, out_bf_ref,
             rs_send, rs_recv, ag_send, ag_recv):
        my_pos = lax.axis_index("i")
        left = (my_pos + N_DEV - 1) % N_DEV
        right = (my_pos + 1) % N_DEV
        diag = (my_pos + 2) % N_DEV
        peers = (right, left, diag)

        barrier_sem = pltpu.get_barrier_semaphore()
        for nbr in peers:
            pl.semaphore_signal(
                barrier_sem, inc=1,
                device_id=(nbr,), device_id_type=pl.DeviceIdType.MESH,
            )
        pl.semaphore_wait(barrier_sem, 3)

        q_ref[...] = jnp.dot(
            x_ref[...], wq_ref[...], preferred_element_type=jnp.float32
        ).astype(jnp.bfloat16)

        def compute_chunk(c):
            b = c // 2
            s0 = (c % 2) * CHUNK
            qi = s0 + lax.broadcasted_iota(jnp.int32, (CHUNK, SQ), 0)
            ki = lax.broadcasted_iota(jnp.int32, (CHUNK, SQ), 1)
            neg = jnp.where(jnp.abs(qi - ki) <= WINDOW, 0.0, -1e9)
            neg = neg.astype(jnp.float32)
            for h in range(HQ_LOCAL):
                q = q_ref[pl.ds(c * CHUNK, CHUNK), h * DH:(h + 1) * DH]
                kv = b * HQ_LOCAL + h
                s = lax.dot_general(
                    q, k_ref[kv], (((1,), (1,)), ((), ())),
                    preferred_element_type=jnp.float32,
                ) * SCALE + neg
                m = jnp.max(s, axis=1, keepdims=True)
                w = jnp.exp(s - m)
                w = (w / jnp.sum(w, axis=1, keepdims=True)).astype(jnp.bfloat16)
                ctx = jnp.dot(w, v_ref[kv], preferred_element_type=jnp.float32)
                accc_ref[:, h * DH:(h + 1) * DH] = ctx.astype(jnp.bfloat16)
            return jnp.dot(accc_ref[...], wo_ref[...],
                           preferred_element_type=jnp.float32)

        rs = []
        for j, p in enumerate(peers):
            sbuf_ref[j] = compute_chunk(p).astype(jnp.bfloat16)
            rdma = pltpu.make_async_remote_copy(
                src_ref=sbuf_ref.at[j],
                dst_ref=rbuf_ref.at[j],
                send_sem=rs_send.at[j],
                recv_sem=rs_recv.at[j],
                device_id=(p,),
                device_id_type=pl.DeviceIdType.MESH,
            )
            rdma.start()
            rs.append(rdma)

        red = compute_chunk(my_pos)
        for rdma in rs:
            rdma.wait()
        for slot in range(3):
            red = red + rbuf_ref[slot].astype(jnp.float32)

        out_ref[pl.ds(my_pos * CHUNK, CHUNK), :] = red
        agbuf_ref[...] = red.astype(jnp.bfloat16)

        ag = []
        for j, p in enumerate(peers):
            rdma = pltpu.make_async_remote_copy(
                src_ref=agbuf_ref,
                dst_ref=out_bf_ref.at[pl.ds(my_pos * CHUNK, CHUNK)],
                send_sem=ag_send.at[j],
                recv_sem=ag_recv.at[j],
                device_id=(p,),
                device_id_type=pl.DeviceIdType.MESH,
            )
            rdma.start()
            ag.append(rdma)
        for j, src_dev in enumerate((left, right, diag)):
            ag[j].wait()
            out_ref[pl.ds(src_dev * CHUNK, CHUNK), :] = out_bf_ref[
                pl.ds(src_dev * CHUNK, CHUNK), :].astype(jnp.float32)

    out = pl.pallas_call(
        body,
        out_shape=jax.ShapeDtypeStruct((ROWS, D_MODEL), jnp.float32),
        in_specs=[pl.BlockSpec(memory_space=pltpu.VMEM)] * 5,
        out_specs=pl.BlockSpec(memory_space=pltpu.VMEM),
        scratch_shapes=[
            pltpu.VMEM((ROWS, QDIM), jnp.bfloat16),
            pltpu.VMEM((CHUNK, QDIM), jnp.bfloat16),
            pltpu.VMEM((3, CHUNK, D_MODEL), jnp.bfloat16),
            pltpu.VMEM((3, CHUNK, D_MODEL), jnp.bfloat16),
            pltpu.VMEM((CHUNK, D_MODEL), jnp.bfloat16),
            pltpu.VMEM((ROWS, D_MODEL), jnp.bfloat16),
            pltpu.SemaphoreType.DMA((3,)),
            pltpu.SemaphoreType.DMA((3,)),
            pltpu.SemaphoreType.DMA((3,)),
            pltpu.SemaphoreType.DMA((3,)),
        ],
        compiler_params=pltpu.CompilerParams(collective_id=0),
    )(x2, Wq_bf, K, V, Wo_bf)

    return out.reshape(B, SQ, D_MODEL)


# device time: 42596 ns/iter; 1.9568x vs baseline; 1.0404x over previous
import jax
import jax.numpy as jnp
from jax import lax
from jax.experimental import pallas as pl
from jax.experimental.pallas import tpu as pltpu

N_DEV = 4
B = 2
SQ = 512
HQ_LOCAL = 8
DH = 64
D_MODEL = 768
QDIM = HQ_LOCAL * DH
ROWS = B * SQ
CHUNK = ROWS // N_DEV
WINDOW = 128
KSPAN = CHUNK + 2 * WINDOW
SCALE = 0.125


def kernel(x, Wq, K_ext, V_ext, Wo):
    my = lax.axis_index("i")
    h0 = my * HQ_LOCAL
    K = lax.dynamic_slice_in_dim(K_ext, h0, HQ_LOCAL, axis=2)
    V = lax.dynamic_slice_in_dim(V_ext, h0, HQ_LOCAL, axis=2)
    K = K.reshape(ROWS, QDIM).astype(jnp.bfloat16)
    V = V.reshape(ROWS, QDIM).astype(jnp.bfloat16)
    x2 = x.reshape(ROWS, D_MODEL).astype(jnp.bfloat16)
    Wq_bf = Wq.astype(jnp.bfloat16)
    Wo_bf = Wo.astype(jnp.bfloat16)

    def body(x_ref, wq_ref, k_ref, v_ref, wo_ref, out_ref,
             q_ref, accc_ref, sbuf_ref, rbuf_ref, agbuf_ref, out_bf_ref,
             rs_send, rs_recv, ag_send, ag_recv):
        my_pos = lax.axis_index("i")
        left = (my_pos + N_DEV - 1) % N_DEV
        right = (my_pos + 1) % N_DEV
        diag = (my_pos + 2) % N_DEV
        peers = (right, left, diag)

        barrier_sem = pltpu.get_barrier_semaphore()
        for nbr in peers:
            pl.semaphore_signal(
                barrier_sem, inc=1,
                device_id=(nbr,), device_id_type=pl.DeviceIdType.MESH,
            )
        pl.semaphore_wait(barrier_sem, 3)

        q_ref[...] = jnp.dot(
            x_ref[...], wq_ref[...], preferred_element_type=jnp.float32
        ).astype(jnp.bfloat16)

        def compute_chunk(c):
            b = c // 2
            s0 = (c % 2) * CHUNK
            k0 = (c % 2) * (CHUNK - WINDOW)
            qi = s0 + lax.broadcasted_iota(jnp.int32, (CHUNK, KSPAN), 0)
            kj = k0 + lax.broadcasted_iota(jnp.int32, (CHUNK, KSPAN), 1)
            neg = jnp.where(jnp.abs(qi - kj) <= WINDOW, 0.0, -1e9)
            neg = neg.astype(jnp.float32)
            krow = b * SQ + k0
            for h in range(HQ_LOCAL):
                q = q_ref[pl.ds(c * CHUNK, CHUNK), h * DH:(h + 1) * DH]
                kslab = k_ref[pl.ds(krow, KSPAN), h * DH:(h + 1) * DH]
                vslab = v_ref[pl.ds(krow, KSPAN), h * DH:(h + 1) * DH]
                s = lax.dot_general(
                    q, kslab, (((1,), (1,)), ((), ())),
                    preferred_element_type=jnp.float32,
                ) * SCALE + neg
                w = jnp.exp(s)
                recip = 1.0 / jnp.sum(w, axis=1, keepdims=True)
                ctx = jnp.dot(w.astype(jnp.bfloat16), vslab,
                              preferred_element_type=jnp.float32) * recip
                accc_ref[:, h * DH:(h + 1) * DH] = ctx.astype(jnp.bfloat16)
            return jnp.dot(accc_ref[...], wo_ref[...],
                           preferred_element_type=jnp.float32)

        rs = []
        for j, p in enumerate(peers):
            sbuf_ref[j] = compute_chunk(p).astype(jnp.bfloat16)
            rdma = pltpu.make_async_remote_copy(
                src_ref=sbuf_ref.at[j],
                dst_ref=rbuf_ref.at[j],
                send_sem=rs_send.at[j],
                recv_sem=rs_recv.at[j],
                device_id=(p,),
                device_id_type=pl.DeviceIdType.MESH,
            )
            rdma.start()
            rs.append(rdma)

        red = compute_chunk(my_pos)
        for rdma in rs:
            rdma.wait()
        for slot in range(3):
            red = red + rbuf_ref[slot].astype(jnp.float32)

        out_ref[pl.ds(my_pos * CHUNK, CHUNK), :] = red
        agbuf_ref[...] = red.astype(jnp.bfloat16)

        ag = []
        for j, p in enumerate(peers):
            rdma = pltpu.make_async_remote_copy(
                src_ref=agbuf_ref,
                dst_ref=out_bf_ref.at[pl.ds(my_pos * CHUNK, CHUNK)],
                send_sem=ag_send.at[j],
                recv_sem=ag_recv.at[j],
                device_id=(p,),
                device_id_type=pl.DeviceIdType.MESH,
            )
            rdma.start()
            ag.append(rdma)
        for j, src_dev in enumerate((left, right, diag)):
            ag[j].wait()
            out_ref[pl.ds(src_dev * CHUNK, CHUNK), :] = out_bf_ref[
                pl.ds(src_dev * CHUNK, CHUNK), :].astype(jnp.float32)

    out = pl.pallas_call(
        body,
        out_shape=jax.ShapeDtypeStruct((ROWS, D_MODEL), jnp.float32),
        in_specs=[pl.BlockSpec(memory_space=pltpu.VMEM)] * 5,
        out_specs=pl.BlockSpec(memory_space=pltpu.VMEM),
        scratch_shapes=[
            pltpu.VMEM((ROWS, QDIM), jnp.bfloat16),
            pltpu.VMEM((CHUNK, QDIM), jnp.bfloat16),
            pltpu.VMEM((3, CHUNK, D_MODEL), jnp.bfloat16),
            pltpu.VMEM((3, CHUNK, D_MODEL), jnp.bfloat16),
            pltpu.VMEM((CHUNK, D_MODEL), jnp.bfloat16),
            pltpu.VMEM((ROWS, D_MODEL), jnp.bfloat16),
            pltpu.SemaphoreType.DMA((3,)),
            pltpu.SemaphoreType.DMA((3,)),
            pltpu.SemaphoreType.DMA((3,)),
            pltpu.SemaphoreType.DMA((3,)),
        ],
        compiler_params=pltpu.CompilerParams(collective_id=0),
    )(x2, Wq_bf, K, V, Wo_bf)

    return out.reshape(B, SQ, D_MODEL)
